# Initial kernel scaffold; baseline (speedup 1.0000x reference)
import jax
import jax.numpy as jnp
from jax.experimental import pallas as pl


def kernel(x, embeddings):
    raise NotImplementedError("write your pallas kernel here")


# speedup vs baseline: 2.0044x; 2.0044x over previous
import jax
import jax.numpy as jnp
from jax.experimental import pallas as pl
from jax.experimental.pallas import tpu as pltpu
from jax.experimental.pallas import tpu_sc as plsc

_TM = 256
_K = 8192
_D = 256
_GATHER_WIN = 128


def _argmin_body(x_ref, xsq_ref, embT_ref, esq_ref, ind_ref):
    xe = jax.lax.dot_general(
        x_ref[...], embT_ref[...],
        dimension_numbers=(((1,), (0,)), ((), ())),
        preferred_element_type=jnp.float32)
    t = (xsq_ref[...] - 2.0 * xe) + esq_ref[...]
    m = jnp.min(t, axis=1, keepdims=True)
    iota = jax.lax.broadcasted_iota(jnp.int32, t.shape, 1)
    arg = jnp.min(jnp.where(t == m, iota, _K), axis=1)
    ind_ref[0, 0, :] = arg


def _compute_indices(xf, xsq, embT, esq):
    n = xf.shape[0]
    grid = n // _TM
    out = pl.pallas_call(
        _argmin_body,
        grid=(grid,),
        in_specs=[
            pl.BlockSpec((_TM, _D), lambda i: (i, 0)),
            pl.BlockSpec((_TM, 1), lambda i: (i, 0)),
            pl.BlockSpec((_D, _K), lambda i: (0, 0)),
            pl.BlockSpec((1, _K), lambda i: (0, 0)),
        ],
        out_specs=pl.BlockSpec((1, 1, _TM), lambda i: (i, 0, 0)),
        out_shape=jax.ShapeDtypeStruct((grid, 1, _TM), jnp.int32),
    )(xf, xsq, embT, esq)
    return out.reshape(n)


def _gather_rows(table, idx):
    n = idx.shape[0]
    d = table.shape[1]
    idx2 = idx.reshape(1, n)
    mesh = plsc.VectorSubcoreMesh(core_axis_name="core",
                                  subcore_axis_name="subcore")

    @pl.kernel(out_type=jax.ShapeDtypeStruct((n, d), table.dtype), mesh=mesh)
    def k(tab_hbm, i_hbm, o_hbm):
        def body(i_vmem, o_vmem):
            pltpu.sync_copy(tab_hbm.at[i_vmem.at[0]], o_vmem)

        pltpu.emit_pipeline(
            body,
            grid=(n // _GATHER_WIN,),
            in_specs=[pl.BlockSpec((1, _GATHER_WIN), index_map=lambda i: (0, i))],
            out_specs=[pl.BlockSpec((_GATHER_WIN, d), index_map=lambda i: (i, 0))],
            core_axis_name="subcore",
            dimension_semantics=(pltpu.PARALLEL,),
        )(i_hbm, o_hbm)

    return k(table, idx2)


def kernel(x, embeddings):
    x = x.astype(jnp.float32)
    xf = x.reshape(-1, x.shape[-1])
    emb = embeddings[0]
    embT = emb.T
    xsq = jnp.sum(xf ** 2, axis=-1, keepdims=True)
    esq = jnp.sum(embeddings ** 2, axis=-1)
    ind = _compute_indices(xf, xsq, embT, esq)
    q = _gather_rows(emb, ind)
    return q.reshape(x.shape), ind.reshape(x.shape[:-1])


# speedup vs baseline: 2.0758x; 1.0356x over previous
import jax
import jax.numpy as jnp
from jax.experimental import pallas as pl
from jax.experimental.pallas import tpu as pltpu
from jax.experimental.pallas import tpu_sc as plsc

_TM = 256
_K = 8192
_D = 256
_GATHER_WIN = 128


def _argmin_body(x_ref, xsq_ref, embT_ref, esq_ref, fio_ref, ind_ref):
    x2 = x_ref[...] + x_ref[...]
    xe2 = jax.lax.dot_general(
        x2, embT_ref[...],
        dimension_numbers=(((1,), (0,)), ((), ())),
        preferred_element_type=jnp.float32)
    t = (xsq_ref[...] - xe2) + esq_ref[...]
    m = jnp.min(t, axis=1, keepdims=True)
    arg = jnp.min(jnp.where(t == m, fio_ref[...], jnp.float32(3.0e38)), axis=1)
    ind_ref[0, 0, :] = arg.astype(jnp.int32)


def _compute_indices(xf, xsq, embT, esq, fio):
    n = xf.shape[0]
    grid = n // _TM
    out = pl.pallas_call(
        _argmin_body,
        grid=(grid,),
        in_specs=[
            pl.BlockSpec((_TM, _D), lambda i: (i, 0)),
            pl.BlockSpec((_TM, 1), lambda i: (i, 0)),
            pl.BlockSpec((_D, _K), lambda i: (0, 0)),
            pl.BlockSpec((1, _K), lambda i: (0, 0)),
            pl.BlockSpec((1, _K), lambda i: (0, 0)),
        ],
        out_specs=pl.BlockSpec((1, 1, _TM), lambda i: (i, 0, 0)),
        out_shape=jax.ShapeDtypeStruct((grid, 1, _TM), jnp.int32),
    )(xf, xsq, embT, esq, fio)
    return out.reshape(n)


def _gather_rows(table, idx):
    n = idx.shape[0]
    d = table.shape[1]
    idx2 = idx.reshape(1, n)
    mesh = plsc.VectorSubcoreMesh(core_axis_name="core",
                                  subcore_axis_name="subcore")

    @pl.kernel(out_type=jax.ShapeDtypeStruct((n, d), table.dtype), mesh=mesh)
    def k(tab_hbm, i_hbm, o_hbm):
        def body(i_vmem, o_vmem):
            pltpu.sync_copy(tab_hbm.at[i_vmem.at[0]], o_vmem)

        pltpu.emit_pipeline(
            body,
            grid=(n // _GATHER_WIN,),
            in_specs=[pl.BlockSpec((1, _GATHER_WIN), index_map=lambda i: (0, i))],
            out_specs=[pl.BlockSpec((_GATHER_WIN, d), index_map=lambda i: (i, 0))],
            core_axis_name="subcore",
            dimension_semantics=(pltpu.PARALLEL,),
        )(i_hbm, o_hbm)

    return k(table, idx2)


def kernel(x, embeddings):
    x = x.astype(jnp.float32)
    xf = x.reshape(-1, x.shape[-1])
    emb = embeddings[0]
    embT = emb.T
    xsq = jnp.sum(xf ** 2, axis=-1, keepdims=True)
    esq = jnp.sum(embeddings ** 2, axis=-1)
    fio = jnp.arange(_K, dtype=jnp.float32)[None, :]
    ind = _compute_indices(xf, xsq, embT, esq, fio)
    q = _gather_rows(emb, ind)
    return q.reshape(x.shape), ind.reshape(x.shape[:-1])


# speedup vs baseline: 2.2030x; 1.0613x over previous
import jax
import jax.numpy as jnp
from jax.experimental import pallas as pl
from jax.experimental.pallas import tpu as pltpu
from jax.experimental.pallas import tpu_sc as plsc

_TM = 512
_K = 8192
_D = 256
_GATHER_WIN = 128


def _argmin_body(x_ref, xsq_ref, embT_ref, esq_ref, fio_ref, ind_ref):
    x2 = x_ref[...] + x_ref[...]
    xe2 = jax.lax.dot_general(
        x2, embT_ref[...],
        dimension_numbers=(((1,), (0,)), ((), ())),
        preferred_element_type=jnp.float32)
    t = (xsq_ref[...] - xe2) + esq_ref[...]
    m = jnp.min(t, axis=1, keepdims=True)
    arg = jnp.min(jnp.where(t == m, fio_ref[...], jnp.float32(3.0e38)), axis=1)
    ind_ref[0, 0, :] = arg.astype(jnp.int32)


def _compute_indices(xf, xsq, embT, esq, fio):
    n = xf.shape[0]
    grid = n // _TM
    out = pl.pallas_call(
        _argmin_body,
        grid=(grid,),
        in_specs=[
            pl.BlockSpec((_TM, _D), lambda i: (i, 0)),
            pl.BlockSpec((_TM, 1), lambda i: (i, 0)),
            pl.BlockSpec((_D, _K), lambda i: (0, 0)),
            pl.BlockSpec((1, _K), lambda i: (0, 0)),
            pl.BlockSpec((1, _K), lambda i: (0, 0)),
        ],
        out_specs=pl.BlockSpec((1, 1, _TM), lambda i: (i, 0, 0)),
        out_shape=jax.ShapeDtypeStruct((grid, 1, _TM), jnp.int32),
    )(xf, xsq, embT, esq, fio)
    return out.reshape(n)


def _gather_rows(table, idx):
    n = idx.shape[0]
    d = table.shape[1]
    idx2 = idx.reshape(1, n)
    mesh = plsc.VectorSubcoreMesh(core_axis_name="core",
                                  subcore_axis_name="subcore")

    @pl.kernel(out_type=jax.ShapeDtypeStruct((n, d), table.dtype), mesh=mesh)
    def k(tab_hbm, i_hbm, o_hbm):
        def body(i_vmem, o_vmem):
            pltpu.sync_copy(tab_hbm.at[i_vmem.at[0]], o_vmem)

        pltpu.emit_pipeline(
            body,
            grid=(n // _GATHER_WIN,),
            in_specs=[pl.BlockSpec((1, _GATHER_WIN), index_map=lambda i: (0, i))],
            out_specs=[pl.BlockSpec((_GATHER_WIN, d), index_map=lambda i: (i, 0))],
            core_axis_name="subcore",
            dimension_semantics=(pltpu.PARALLEL,),
        )(i_hbm, o_hbm)

    return k(table, idx2)


def kernel(x, embeddings):
    x = x.astype(jnp.float32)
    xf = x.reshape(-1, x.shape[-1])
    emb = embeddings[0]
    embT = emb.T
    xsq = jnp.sum(xf ** 2, axis=-1, keepdims=True)
    esq = jnp.sum(embeddings ** 2, axis=-1)
    fio = jnp.arange(_K, dtype=jnp.float32)[None, :]
    ind = _compute_indices(xf, xsq, embT, esq, fio)
    q = _gather_rows(emb, ind)
    return q.reshape(x.shape), ind.reshape(x.shape[:-1])


# speedup vs baseline: 2.3929x; 1.0862x over previous
import jax
import jax.numpy as jnp
from jax.experimental import pallas as pl
from jax.experimental.pallas import tpu as pltpu
from jax.experimental.pallas import tpu_sc as plsc

_TM = 512
_RB = 128
_K = 8192
_D = 256
_GATHER_WIN = 128


def _argmin_body(x_ref, xsq_ref, embT_ref, esq_ref, fio_ref, ind_ref):
    x2 = x_ref[...] + x_ref[...]
    xe2 = jax.lax.dot_general(
        x2, embT_ref[...],
        dimension_numbers=(((1,), (0,)), ((), ())),
        preferred_element_type=jnp.float32)
    esq = esq_ref[...]
    lane = fio_ref[0:1, 0:128]
    for r0 in range(0, _TM, _RB):
        rows = slice(r0, r0 + _RB)
        xsq_r = xsq_ref[rows, :]
        M = (xsq_r - xe2[rows, 0:128]) + esq[:, 0:128]
        G = jnp.zeros((_RB, 128), jnp.float32)
        for g in range(1, _K // 128):
            sl = slice(g * 128, (g + 1) * 128)
            t = (xsq_r - xe2[rows, sl]) + esq[:, sl]
            lt = t < M
            M = jnp.where(lt, t, M)
            G = jnp.where(lt, jnp.float32(g), G)
        k_idx = G * 128.0 + lane
        m = jnp.min(M, axis=1, keepdims=True)
        cand = jnp.where(M == m, k_idx, jnp.float32(3.0e38))
        arg = jnp.min(cand, axis=1)
        ind_ref[0, 0, r0:r0 + _RB] = arg.astype(jnp.int32)


def _compute_indices(xf, xsq, embT, esq, fio):
    n = xf.shape[0]
    grid = n // _TM
    out = pl.pallas_call(
        _argmin_body,
        grid=(grid,),
        in_specs=[
            pl.BlockSpec((_TM, _D), lambda i: (i, 0)),
            pl.BlockSpec((_TM, 1), lambda i: (i, 0)),
            pl.BlockSpec((_D, _K), lambda i: (0, 0)),
            pl.BlockSpec((1, _K), lambda i: (0, 0)),
            pl.BlockSpec((1, _K), lambda i: (0, 0)),
        ],
        out_specs=pl.BlockSpec((1, 1, _TM), lambda i: (i, 0, 0)),
        out_shape=jax.ShapeDtypeStruct((grid, 1, _TM), jnp.int32),
    )(xf, xsq, embT, esq, fio)
    return out.reshape(n)


def _gather_rows(table, idx):
    n = idx.shape[0]
    d = table.shape[1]
    idx2 = idx.reshape(1, n)
    mesh = plsc.VectorSubcoreMesh(core_axis_name="core",
                                  subcore_axis_name="subcore")

    @pl.kernel(out_type=jax.ShapeDtypeStruct((n, d), table.dtype), mesh=mesh)
    def k(tab_hbm, i_hbm, o_hbm):
        def body(i_vmem, o_vmem):
            pltpu.sync_copy(tab_hbm.at[i_vmem.at[0]], o_vmem)

        pltpu.emit_pipeline(
            body,
            grid=(n // _GATHER_WIN,),
            in_specs=[pl.BlockSpec((1, _GATHER_WIN), index_map=lambda i: (0, i))],
            out_specs=[pl.BlockSpec((_GATHER_WIN, d), index_map=lambda i: (i, 0))],
            core_axis_name="subcore",
            dimension_semantics=(pltpu.PARALLEL,),
        )(i_hbm, o_hbm)

    return k(table, idx2)


def kernel(x, embeddings):
    x = x.astype(jnp.float32)
    xf = x.reshape(-1, x.shape[-1])
    emb = embeddings[0]
    embT = emb.T
    xsq = jnp.sum(xf ** 2, axis=-1, keepdims=True)
    esq = jnp.sum(embeddings ** 2, axis=-1)
    fio = jnp.arange(_K, dtype=jnp.float32)[None, :]
    ind = _compute_indices(xf, xsq, embT, esq, fio)
    q = _gather_rows(emb, ind)
    return q.reshape(x.shape), ind.reshape(x.shape[:-1])


# speedup vs baseline: 2.4672x; 1.0311x over previous
import jax
import jax.numpy as jnp
from jax.experimental import pallas as pl
from jax.experimental.pallas import tpu as pltpu
from jax.experimental.pallas import tpu_sc as plsc

_TM = 512
_RB = 128
_K = 8192
_D = 256
_GATHER_WIN = 128


def _argmin_body(x_ref, xsq_ref, embT_ref, esq_ref, fio_ref, ind_ref):
    x2 = x_ref[...] + x_ref[...]
    xe2 = jax.lax.dot_general(
        x2, embT_ref[...],
        dimension_numbers=(((1,), (0,)), ((), ())),
        preferred_element_type=jnp.float32)
    esq = esq_ref[...]
    lane = fio_ref[0:1, 0:128]
    for r0 in range(0, _TM, _RB):
        rows = slice(r0, r0 + _RB)
        xsq_r = xsq_ref[rows, :]
        M = (xsq_r - xe2[rows, 0:128]) + esq[:, 0:128]
        G = jnp.zeros((_RB, 128), jnp.float32)
        for g in range(1, _K // 128):
            sl = slice(g * 128, (g + 1) * 128)
            t = (xsq_r - xe2[rows, sl]) + esq[:, sl]
            lt = t < M
            M = jnp.where(lt, t, M)
            G = jnp.where(lt, jnp.float32(g), G)
        k_idx = G * 128.0 + lane
        m = jnp.min(M, axis=1, keepdims=True)
        cand = jnp.where(M == m, k_idx, jnp.float32(3.0e38))
        arg = jnp.min(cand, axis=1)
        ind_ref[0, 0, r0:r0 + _RB] = arg.astype(jnp.int32)


def _compute_indices(xf, xsq, embT, esq, fio):
    n = xf.shape[0]
    grid = n // _TM
    out = pl.pallas_call(
        _argmin_body,
        grid=(grid,),
        in_specs=[
            pl.BlockSpec((_TM, _D), lambda i: (i, 0)),
            pl.BlockSpec((_TM, 1), lambda i: (i, 0)),
            pl.BlockSpec((_D, _K), lambda i: (0, 0)),
            pl.BlockSpec((1, _K), lambda i: (0, 0)),
            pl.BlockSpec((1, _K), lambda i: (0, 0)),
        ],
        out_specs=pl.BlockSpec((1, 1, _TM), lambda i: (i, 0, 0)),
        out_shape=jax.ShapeDtypeStruct((grid, 1, _TM), jnp.int32),
    )(xf, xsq, embT, esq, fio)
    return out.reshape(n)


def _gather_rows(table, idx):
    n = idx.shape[0]
    d = table.shape[1]
    idx2 = idx.reshape(1, n)
    mesh = plsc.VectorSubcoreMesh(core_axis_name="core",
                                  subcore_axis_name="subcore")

    @pl.kernel(out_type=jax.ShapeDtypeStruct((n, d), table.dtype), mesh=mesh)
    def k(tab_hbm, i_hbm, o_hbm):
        def body(i_vmem, o_vmem):
            pltpu.sync_copy(tab_hbm.at[i_vmem.at[0]], o_vmem)

        pltpu.emit_pipeline(
            body,
            grid=(n // _GATHER_WIN,),
            in_specs=[pl.BlockSpec((1, _GATHER_WIN), index_map=lambda i: (0, i))],
            out_specs=[pl.BlockSpec((_GATHER_WIN, d), index_map=lambda i: (i, 0))],
            core_axis_name="subcore",
            dimension_semantics=(pltpu.PARALLEL,),
        )(i_hbm, o_hbm)

    return k(table, idx2)


def kernel(x, embeddings):
    x = x.astype(jnp.float32)
    xf = x.reshape(-1, x.shape[-1])
    emb = embeddings[0]
    embT = emb.T.astype(jnp.bfloat16)
    xsq = jnp.sum(xf ** 2, axis=-1, keepdims=True)
    esq = jnp.sum(embeddings ** 2, axis=-1)
    fio = jnp.arange(_K, dtype=jnp.float32)[None, :]
    ind = _compute_indices(xf, xsq, embT, esq, fio)
    q = _gather_rows(emb, ind)
    return q.reshape(x.shape), ind.reshape(x.shape[:-1])


# speedup vs baseline: 2.4934x; 1.0106x over previous
import jax
import jax.numpy as jnp
from jax.experimental import pallas as pl
from jax.experimental.pallas import tpu as pltpu
from jax.experimental.pallas import tpu_sc as plsc

_TM = 512
_RB = 128
_KC = 1024
_K = 8192
_D = 256
_GATHER_WIN = 128


def _argmin_body(x_ref, xsq_ref, embT_ref, esq_ref, fio_ref, ind_ref):
    esq = esq_ref[...]
    lane = fio_ref[0:1, 0:128]
    for r0 in range(0, _TM, _RB):
        rows = slice(r0, r0 + _RB)
        x2_r = x_ref[rows, :] + x_ref[rows, :]
        xsq_r = xsq_ref[rows, :]
        M = None
        G = None
        for c0 in range(0, _K, _KC):
            xe = jax.lax.dot_general(
                x2_r, embT_ref[:, c0:c0 + _KC],
                dimension_numbers=(((1,), (0,)), ((), ())),
                preferred_element_type=jnp.float32)
            for g0 in range(0, _KC, 128):
                t = (xsq_r - xe[:, g0:g0 + 128]) + esq[:, c0 + g0:c0 + g0 + 128]
                if M is None:
                    M = t
                    G = jnp.zeros((_RB, 128), jnp.float32)
                else:
                    lt = t < M
                    M = jnp.where(lt, t, M)
                    G = jnp.where(lt, jnp.float32((c0 + g0) // 128), G)
        k_idx = G * 128.0 + lane
        m = jnp.min(M, axis=1, keepdims=True)
        cand = jnp.where(M == m, k_idx, jnp.float32(3.0e38))
        arg = jnp.min(cand, axis=1)
        ind_ref[0, 0, r0:r0 + _RB] = arg.astype(jnp.int32)


def _compute_indices(xf, xsq, embT, esq, fio):
    n = xf.shape[0]
    grid = n // _TM
    out = pl.pallas_call(
        _argmin_body,
        grid=(grid,),
        in_specs=[
            pl.BlockSpec((_TM, _D), lambda i: (i, 0)),
            pl.BlockSpec((_TM, 1), lambda i: (i, 0)),
            pl.BlockSpec((_D, _K), lambda i: (0, 0)),
            pl.BlockSpec((1, _K), lambda i: (0, 0)),
            pl.BlockSpec((1, _K), lambda i: (0, 0)),
        ],
        out_specs=pl.BlockSpec((1, 1, _TM), lambda i: (i, 0, 0)),
        out_shape=jax.ShapeDtypeStruct((grid, 1, _TM), jnp.int32),
    )(xf, xsq, embT, esq, fio)
    return out.reshape(n)


def _gather_rows(table, idx):
    n = idx.shape[0]
    d = table.shape[1]
    idx2 = idx.reshape(1, n)
    mesh = plsc.VectorSubcoreMesh(core_axis_name="core",
                                  subcore_axis_name="subcore")

    @pl.kernel(out_type=jax.ShapeDtypeStruct((n, d), table.dtype), mesh=mesh)
    def k(tab_hbm, i_hbm, o_hbm):
        def body(i_vmem, o_vmem):
            pltpu.sync_copy(tab_hbm.at[i_vmem.at[0]], o_vmem)

        pltpu.emit_pipeline(
            body,
            grid=(n // _GATHER_WIN,),
            in_specs=[pl.BlockSpec((1, _GATHER_WIN), index_map=lambda i: (0, i))],
            out_specs=[pl.BlockSpec((_GATHER_WIN, d), index_map=lambda i: (i, 0))],
            core_axis_name="subcore",
            dimension_semantics=(pltpu.PARALLEL,),
        )(i_hbm, o_hbm)

    return k(table, idx2)


def kernel(x, embeddings):
    x = x.astype(jnp.float32)
    xf = x.reshape(-1, x.shape[-1])
    emb = embeddings[0]
    embT = emb.T.astype(jnp.bfloat16)
    xsq = jnp.sum(xf ** 2, axis=-1, keepdims=True)
    esq = jnp.sum(embeddings ** 2, axis=-1)
    fio = jnp.arange(_K, dtype=jnp.float32)[None, :]
    ind = _compute_indices(xf, xsq, embT, esq, fio)
    q = _gather_rows(emb, ind)
    return q.reshape(x.shape), ind.reshape(x.shape[:-1])


# speedup vs baseline: 2.6710x; 1.0712x over previous
import jax
import jax.numpy as jnp
from jax.experimental import pallas as pl
from jax.experimental.pallas import tpu as pltpu
from jax.experimental.pallas import tpu_sc as plsc

_TM = 512
_RB = 128
_KC = 1024
_K = 8192
_D = 256
_GATHER_WIN = 128


def _argmin_body(x_ref, xsq_ref, embT_ref, esq_ref, fio_ref, ind_ref):
    esq = esq_ref[...]
    lane = fio_ref[0:1, 0:128]
    for r0 in range(0, _TM, _RB):
        rows = slice(r0, r0 + _RB)
        x2_r = x_ref[rows, :] + x_ref[rows, :]
        xsq_r = xsq_ref[rows, :]
        M = None
        G = None
        for c0 in range(0, _K, _KC):
            xe = jax.lax.dot_general(
                x2_r, embT_ref[:, c0:c0 + _KC],
                dimension_numbers=(((1,), (0,)), ((), ())),
                preferred_element_type=jnp.float32)
            for g0 in range(0, _KC, 128):
                t = (xsq_r - xe[:, g0:g0 + 128]) + esq[:, c0 + g0:c0 + g0 + 128]
                if M is None:
                    M = t
                    G = jnp.zeros((_RB, 128), jnp.float32)
                else:
                    lt = t < M
                    M = jnp.where(lt, t, M)
                    G = jnp.where(lt, jnp.float32((c0 + g0) // 128), G)
        k_idx = G * 128.0 + lane
        m = jnp.min(M, axis=1, keepdims=True)
        cand = jnp.where(M == m, k_idx, jnp.float32(3.0e38))
        arg = jnp.min(cand, axis=1)
        ind_ref[0, 0, r0:r0 + _RB] = arg.astype(jnp.int32)


def _compute_indices(xf, xsq, embT, esq, fio):
    n = xf.shape[0]
    grid = n // _TM
    out = pl.pallas_call(
        _argmin_body,
        grid=(grid,),
        in_specs=[
            pl.BlockSpec((_TM, _D), lambda i: (i, 0)),
            pl.BlockSpec((_TM, 1), lambda i: (i, 0)),
            pl.BlockSpec((_D, _K), lambda i: (0, 0)),
            pl.BlockSpec((1, _K), lambda i: (0, 0)),
            pl.BlockSpec((1, _K), lambda i: (0, 0)),
        ],
        out_specs=pl.BlockSpec((1, 1, _TM), lambda i: (i, 0, 0)),
        out_shape=jax.ShapeDtypeStruct((grid, 1, _TM), jnp.int32),
    )(xf, xsq, embT, esq, fio)
    return out.reshape(n)


def _gather_rows(table, idx):
    n = idx.shape[0]
    d = table.shape[1]
    idx2 = idx.reshape(1, n)
    mesh = plsc.VectorSubcoreMesh(core_axis_name="core",
                                  subcore_axis_name="subcore")

    @pl.kernel(out_type=jax.ShapeDtypeStruct((n, d), table.dtype), mesh=mesh)
    def k(tab_hbm, i_hbm, o_hbm):
        def body(i_vmem, o_vmem):
            pltpu.sync_copy(tab_hbm.at[i_vmem.at[0]], o_vmem)

        pltpu.emit_pipeline(
            body,
            grid=(n // _GATHER_WIN,),
            in_specs=[pl.BlockSpec((1, _GATHER_WIN), index_map=lambda i: (0, i))],
            out_specs=[pl.BlockSpec((_GATHER_WIN, d), index_map=lambda i: (i, 0))],
            core_axis_name=("core", "subcore"),
            dimension_semantics=(pltpu.PARALLEL,),
        )(i_hbm, o_hbm)

    return k(table, idx2)


def kernel(x, embeddings):
    x = x.astype(jnp.float32)
    xf = x.reshape(-1, x.shape[-1])
    emb = embeddings[0]
    embT = emb.T.astype(jnp.bfloat16)
    xsq = jnp.sum(xf ** 2, axis=-1, keepdims=True)
    esq = jnp.sum(embeddings ** 2, axis=-1)
    fio = jnp.arange(_K, dtype=jnp.float32)[None, :]
    ind = _compute_indices(xf, xsq, embT, esq, fio)
    q = _gather_rows(emb, ind)
    return q.reshape(x.shape), ind.reshape(x.shape[:-1])
